# Initial kernel scaffold; baseline (speedup 1.0000x reference)
import jax
import jax.numpy as jnp
from jax.experimental import pallas as pl


def kernel(features, edges, W_self_0, W_neigh_0, b_0, gamma_0, beta_0, W_self_1, W_neigh_1, b_1, gamma_1, beta_1, W_self_2, W_neigh_2, b_2, gamma_2, beta_2):
    raise NotImplementedError("write your pallas kernel here")


# speedup vs baseline: 6.3564x; 6.3564x over previous
import functools

import jax
import jax.numpy as jnp
from jax import lax
from jax.experimental import pallas as pl
from jax.experimental.pallas import tpu as pltpu
from jax.experimental.pallas import tpu_sc as plsc

_N = 10000
_E = 320000
_D = 128

_NC = 2
_NS = 16
_NW = _NC * _NS
_CH = 80
_NCHUNK = _E // (_NW * _CH)
_RSTEP = 624
_RCNT = 640


def _sc_segment_sum(xn, src_w, dst_w, zrows):
    mesh = plsc.VectorSubcoreMesh(core_axis_name="c", subcore_axis_name="s")

    @functools.partial(
        pl.kernel,
        out_type=jax.ShapeDtypeStruct((_NC, _N, _D), jnp.float32),
        mesh=mesh,
        scratch_types=[
            pltpu.VMEM_SHARED((_N, _D), jnp.float32),
            pltpu.VMEM((_NCHUNK, _CH), jnp.int32),
            pltpu.VMEM((_NCHUNK, _CH), jnp.int32),
            pltpu.VMEM((_CH, _D), jnp.float32),
            pltpu.SemaphoreType.DMA,
        ],
    )
    def seg_sum(xn_hbm, src_hbm, dst_hbm, z_hbm, out_hbm,
                acc, src_v, dst_v, rows_v, sem):
        c = lax.axis_index("c")
        s = lax.axis_index("s")
        wid = s * _NC + c
        pltpu.sync_copy(z_hbm, acc.at[pl.ds(s * _RSTEP, _RCNT)])
        pltpu.sync_copy(src_hbm.at[wid], src_v)
        pltpu.sync_copy(dst_hbm.at[wid], dst_v)
        plsc.subcore_barrier()

        def body(j, carry):
            pltpu.async_copy(xn_hbm.at[src_v.at[j]], rows_v, sem).wait()
            pltpu.sync_copy(rows_v, acc.at[dst_v.at[j]], add=True)
            return carry

        lax.fori_loop(0, _NCHUNK, body, 0)
        plsc.subcore_barrier()
        pltpu.sync_copy(acc.at[pl.ds(s * _RSTEP, _RCNT)],
                        out_hbm.at[c, pl.ds(s * _RSTEP, _RCNT)])

    return seg_sum(xn, src_w, dst_w, zrows)


_R = 2000


def _dot(a, b):
    return lax.dot_general(a, b, (((1,), (0,)), ((), ())),
                           precision=lax.Precision.HIGHEST,
                           preferred_element_type=jnp.float32)


def _tc_matmul(x, w):
    def body(x_ref, w_ref, o_ref):
        o_ref[...] = _dot(x_ref[...], w_ref[...])

    return pl.pallas_call(
        body,
        grid=(_N // _R,),
        in_specs=[pl.BlockSpec((_R, _D), lambda i: (i, 0)),
                  pl.BlockSpec((_D, _D), lambda i: (0, 0))],
        out_specs=pl.BlockSpec((_R, _D), lambda i: (i, 0)),
        out_shape=jax.ShapeDtypeStruct((_N, _D), jnp.float32),
    )(x, w)


def _tc_combine(x, agg2, w_self, b, gamma, beta, w_next):
    has_next = w_next is not None

    def body(x_ref, agg_ref, ws_ref, b_ref, g_ref, be_ref, *rest):
        if has_next:
            wn_ref, h_ref, y_ref = rest
        else:
            (h_ref,) = rest
        t = (_dot(x_ref[...], ws_ref[...])
             + agg_ref[0] + agg_ref[1] + b_ref[...])
        mu = jnp.mean(t, axis=-1, keepdims=True)
        d = t - mu
        var = jnp.mean(d * d, axis=-1, keepdims=True)
        h = d * lax.rsqrt(var + 1e-5) * g_ref[...] + be_ref[...]
        h = jnp.maximum(h, 0.0)
        h_ref[...] = h
        if has_next:
            y_ref[...] = _dot(h, wn_ref[...])

    row_spec = pl.BlockSpec((_R, _D), lambda i: (i, 0))
    full_spec = pl.BlockSpec((_D, _D), lambda i: (0, 0))
    vec_spec = pl.BlockSpec((1, _D), lambda i: (0, 0))
    in_specs = [row_spec,
                pl.BlockSpec((2, _R, _D), lambda i: (0, i, 0)),
                full_spec, vec_spec, vec_spec, vec_spec]
    args = [x, agg2, w_self, b, gamma, beta]
    out_shape = jax.ShapeDtypeStruct((_N, _D), jnp.float32)
    if has_next:
        in_specs.append(full_spec)
        args.append(w_next)
        return pl.pallas_call(
            body,
            grid=(_N // _R,),
            in_specs=in_specs,
            out_specs=(row_spec, row_spec),
            out_shape=(out_shape, out_shape),
        )(*args)
    return pl.pallas_call(
        body,
        grid=(_N // _R,),
        in_specs=in_specs,
        out_specs=row_spec,
        out_shape=out_shape,
    )(*args)


def kernel(features, edges,
           W_self_0, W_neigh_0, b_0, gamma_0, beta_0,
           W_self_1, W_neigh_1, b_1, gamma_1, beta_1,
           W_self_2, W_neigh_2, b_2, gamma_2, beta_2):
    src_w = edges[0].reshape(_NW, _NCHUNK, _CH)
    dst_w = edges[1].reshape(_NW, _NCHUNK, _CH)
    zrows = jnp.zeros((_RCNT, _D), jnp.float32)
    b0, g0, be0 = b_0.reshape(1, _D), gamma_0.reshape(1, _D), beta_0.reshape(1, _D)
    b1, g1, be1 = b_1.reshape(1, _D), gamma_1.reshape(1, _D), beta_1.reshape(1, _D)
    b2, g2, be2 = b_2.reshape(1, _D), gamma_2.reshape(1, _D), beta_2.reshape(1, _D)

    xn0 = _tc_matmul(features, W_neigh_0)
    agg0 = _sc_segment_sum(xn0, src_w, dst_w, zrows)
    h1, xn1 = _tc_combine(features, agg0, W_self_0, b0, g0, be0, W_neigh_1)
    agg1 = _sc_segment_sum(xn1, src_w, dst_w, zrows)
    h2, xn2 = _tc_combine(h1, agg1, W_self_1, b1, g1, be1, W_neigh_2)
    agg2 = _sc_segment_sum(xn2, src_w, dst_w, zrows)
    return _tc_combine(h2, agg2, W_self_2, b2, g2, be2, None)


# speedup vs baseline: 9.3428x; 1.4698x over previous
import functools

import jax
import jax.numpy as jnp
from jax import lax
from jax.experimental import pallas as pl
from jax.experimental.pallas import tpu as pltpu
from jax.experimental.pallas import tpu_sc as plsc

_N = 10000
_E = 320000
_D = 128

_NC = 2
_NS = 16
_NW = _NC * _NS
_CH = 128
_EPW = 10240
_NCHUNK = _EPW // _CH
_NPAD = _EPW - _E // _NW
_NDUMMY = 16
_NA = _N + _NDUMMY
_RSTEP = 624
_RCNT = 640


def _sc_segment_sum(xn, src_w, dstp_w, zrows):
    mesh = plsc.VectorSubcoreMesh(core_axis_name="c", subcore_axis_name="s")

    @functools.partial(
        pl.kernel,
        out_type=jax.ShapeDtypeStruct((_NC, _N, _D), jnp.float32),
        mesh=mesh,
        scratch_types=[
            pltpu.VMEM_SHARED((_NA, _D), jnp.float32),
            pltpu.VMEM((_NCHUNK, _CH), jnp.int32),
            pltpu.VMEM((_NCHUNK * _CH // 2,), jnp.int32),
            pltpu.VMEM((8, _CH), jnp.int32),
            pltpu.VMEM((_CH, _D), jnp.float32),
            pltpu.VMEM((_CH, _D), jnp.float32),
            pltpu.SemaphoreType.DMA,
            pltpu.SemaphoreType.DMA,
        ],
    )
    def seg_sum(xn_hbm, src_hbm, dstp_hbm, z_hbm, out_hbm,
                acc, src_v, dstp_v, scat_v, rows_a, rows_b, sem_a, sem_b):
        c = lax.axis_index("c")
        s = lax.axis_index("s")
        wid = s * _NC + c
        pltpu.sync_copy(z_hbm, acc.at[pl.ds(s * _RSTEP, _RCNT)])
        pltpu.sync_copy(src_hbm.at[wid], src_v)
        pltpu.sync_copy(dstp_hbm.at[wid], dstp_v)
        plsc.subcore_barrier()

        def unpack_dst(j):
            base = j * (_CH // 2)
            for t in range(_CH // 32):
                v = dstp_v[pl.ds(base + t * 16, 16)]
                scat_v[0, pl.ds(t * 16, 16)] = v & 0xFFFF
                scat_v[0, pl.ds(_CH // 2 + t * 16, 16)] = lax.shift_right_logical(v, 16)

        def scatter_add(rows):
            pltpu.sync_copy(rows, acc.at[scat_v.at[0]], add=True)

        def gather(j, rows, sem):
            pltpu.async_copy(xn_hbm.at[src_v.at[j]], rows, sem)

        def two_chunks(j, issue_next):
            pltpu.make_async_copy(xn_hbm.at[src_v.at[j]], rows_a, sem_a).wait()
            gather(j + 1, rows_b, sem_b)
            unpack_dst(j)
            scatter_add(rows_a)
            pltpu.make_async_copy(xn_hbm.at[src_v.at[j + 1]], rows_b, sem_b).wait()
            if issue_next:
                gather(j + 2, rows_a, sem_a)
            unpack_dst(j + 1)
            scatter_add(rows_b)

        gather(0, rows_a, sem_a)

        def body(i, carry):
            two_chunks(2 * i, True)
            return carry

        lax.fori_loop(0, (_NCHUNK - 2) // 2, body, 0)
        two_chunks(_NCHUNK - 2, False)
        plsc.subcore_barrier()
        pltpu.sync_copy(acc.at[pl.ds(s * _RSTEP, _RCNT)],
                        out_hbm.at[c, pl.ds(s * _RSTEP, _RCNT)])

    return seg_sum(xn, src_w, dstp_w, zrows)


_R = 2000


def _dot(a, b):
    return lax.dot_general(a, b, (((1,), (0,)), ((), ())),
                           precision=lax.Precision.HIGHEST,
                           preferred_element_type=jnp.float32)


def _tc_matmul(x, w):
    def body(x_ref, w_ref, o_ref):
        o_ref[...] = _dot(x_ref[...], w_ref[...])

    return pl.pallas_call(
        body,
        grid=(_N // _R,),
        in_specs=[pl.BlockSpec((_R, _D), lambda i: (i, 0)),
                  pl.BlockSpec((_D, _D), lambda i: (0, 0))],
        out_specs=pl.BlockSpec((_R, _D), lambda i: (i, 0)),
        out_shape=jax.ShapeDtypeStruct((_N, _D), jnp.float32),
    )(x, w)


def _tc_combine(x, agg2, w_self, b, gamma, beta, w_next):
    has_next = w_next is not None

    def body(x_ref, agg_ref, ws_ref, b_ref, g_ref, be_ref, *rest):
        if has_next:
            wn_ref, h_ref, y_ref = rest
        else:
            (h_ref,) = rest
        t = (_dot(x_ref[...], ws_ref[...])
             + agg_ref[0] + agg_ref[1] + b_ref[...])
        mu = jnp.mean(t, axis=-1, keepdims=True)
        d = t - mu
        var = jnp.mean(d * d, axis=-1, keepdims=True)
        h = d * lax.rsqrt(var + 1e-5) * g_ref[...] + be_ref[...]
        h = jnp.maximum(h, 0.0)
        h_ref[...] = h
        if has_next:
            y_ref[...] = _dot(h, wn_ref[...])

    row_spec = pl.BlockSpec((_R, _D), lambda i: (i, 0))
    full_spec = pl.BlockSpec((_D, _D), lambda i: (0, 0))
    vec_spec = pl.BlockSpec((1, _D), lambda i: (0, 0))
    in_specs = [row_spec,
                pl.BlockSpec((2, _R, _D), lambda i: (0, i, 0)),
                full_spec, vec_spec, vec_spec, vec_spec]
    args = [x, agg2, w_self, b, gamma, beta]
    out_shape = jax.ShapeDtypeStruct((_N, _D), jnp.float32)
    if has_next:
        in_specs.append(full_spec)
        args.append(w_next)
        return pl.pallas_call(
            body,
            grid=(_N // _R,),
            in_specs=in_specs,
            out_specs=(row_spec, row_spec),
            out_shape=(out_shape, out_shape),
        )(*args)
    return pl.pallas_call(
        body,
        grid=(_N // _R,),
        in_specs=in_specs,
        out_specs=row_spec,
        out_shape=out_shape,
    )(*args)


def kernel(features, edges,
           W_self_0, W_neigh_0, b_0, gamma_0, beta_0,
           W_self_1, W_neigh_1, b_1, gamma_1, beta_1,
           W_self_2, W_neigh_2, b_2, gamma_2, beta_2):
    i_pad = lax.broadcasted_iota(jnp.int32, (_NW, _NPAD), 1)
    w_pad = lax.broadcasted_iota(jnp.int32, (_NW, _NPAD), 0)
    pad_src = (w_pad * 997 + i_pad * 13) % _N
    pad_dst = _N + (i_pad % _NDUMMY)
    src_w = jnp.concatenate(
        [edges[0].reshape(_NW, _E // _NW), pad_src], axis=1
    ).reshape(_NW, _NCHUNK, _CH)
    dst3 = jnp.concatenate(
        [edges[1].reshape(_NW, _E // _NW), pad_dst], axis=1
    ).reshape(_NW, _NCHUNK, _CH)
    dstp_w = (dst3[:, :, : _CH // 2] | (dst3[:, :, _CH // 2:] << 16)
              ).reshape(_NW, _NCHUNK * _CH // 2)
    zrows = jnp.zeros((_RCNT, _D), jnp.float32)
    b0, g0, be0 = b_0.reshape(1, _D), gamma_0.reshape(1, _D), beta_0.reshape(1, _D)
    b1, g1, be1 = b_1.reshape(1, _D), gamma_1.reshape(1, _D), beta_1.reshape(1, _D)
    b2, g2, be2 = b_2.reshape(1, _D), gamma_2.reshape(1, _D), beta_2.reshape(1, _D)

    xn0 = _tc_matmul(features, W_neigh_0)
    agg0 = _sc_segment_sum(xn0, src_w, dstp_w, zrows)
    h1, xn1 = _tc_combine(features, agg0, W_self_0, b0, g0, be0, W_neigh_1)
    agg1 = _sc_segment_sum(xn1, src_w, dstp_w, zrows)
    h2, xn2 = _tc_combine(h1, agg1, W_self_1, b1, g1, be1, W_neigh_2)
    agg2 = _sc_segment_sum(xn2, src_w, dstp_w, zrows)
    return _tc_combine(h2, agg2, W_self_2, b2, g2, be2, None)
